# baseline (speedup 1.0000x reference)
import functools

import jax
import jax.numpy as jnp
from jax import lax
from jax.experimental import pallas as pl
from jax.experimental.pallas import tpu as pltpu
from jax.experimental.pallas import tpu_sc as plsc

_N = 10000
_E = 160000
_D_IN = 128
_D_HID = 256
_D_OUT = 128
_DH = 64
_NC, _NS = 2, 16
_NP = 10240
_RPT = _NP // _NS
_EPT = _E // _NS
_CH = 125
_NCHUNK = _EPT // _CH
_DEGW = 16


def _spmm_body(xlo, xhi, r0, c0, r1, c1, z64, zdeg,
               o0lo, o0hi, o1lo, o1hi, dg0, dg1,
               idxr0, idxc0, idxr1, idxc1, gb0, gb1, gb2, gb3,
               ones, acc, dacc, gsemA, gsemB, ssemA, ssemB):
    cid = lax.axis_index("c")
    sid = lax.axis_index("s")
    gbufs = (gb0, gb1, gb2, gb3)
    rows = pl.ds(sid * _RPT, _RPT)

    def _ones_row(j, carry):
        ones[j, :] = jnp.ones((16,), jnp.float32)
        return carry
    lax.fori_loop(0, _CH, _ones_row, 0)

    pltpu.sync_copy(z64.at[rows], acc.at[rows])
    pltpu.sync_copy(zdeg.at[rows], dacc.at[rows])

    echunks = pl.ds(sid * _NCHUNK, _NCHUNK)
    pltpu.sync_copy(r0.at[echunks], idxr0)
    pltpu.sync_copy(c0.at[echunks], idxc0)
    pltpu.sync_copy(r1.at[echunks], idxr1)
    pltpu.sync_copy(c1.at[echunks], idxc1)
    plsc.subcore_barrier()

    _NR = _NCHUNK // 2
    for set_idx, (idxr, idxc, olo, ohi) in enumerate(
            ((idxr0, idxc0, o0lo, o0hi), (idxr1, idxc1, o1lo, o1hi))):
        for half, xh in ((0, xlo), (1, xhi)):
            @pl.when(cid == half)
            def _(xh=xh, idxr=idxr, idxc=idxc, set_idx=set_idx):
                quads = ((gbufs[:2], gsemA, ssemA), (gbufs[2:], gsemB, ssemB))
                for b in range(2):
                    pltpu.async_copy(xh.at[idxc.at[b]], quads[0][0][b], gsemA)

                def _round(r, carry):
                    base = 2 * r
                    for q in (0, 1):
                        @pl.when(lax.rem(r, 2) == q)
                        def _(q=q):
                            bufs, gsem, ssem = quads[q]
                            obufs, ogsem, ossem = quads[1 - q]
                            for b in range(2):
                                pltpu.make_async_copy(
                                    xh.at[idxc.at[base + b]], bufs[b],
                                    gsem).wait()
                            for b in range(2):
                                pltpu.async_copy(
                                    bufs[b], acc.at[idxr.at[base + b]],
                                    ssem, add=True)

                            @pl.when(cid == set_idx)
                            def _():
                                for b in range(2):
                                    pltpu.sync_copy(
                                        ones, dacc.at[idxr.at[base + b]],
                                        add=True)

                            @pl.when(r >= 1)
                            def _(obufs=obufs, ossem=ossem):
                                for b in range(2):
                                    pltpu.make_async_copy(
                                        obufs[b],
                                        acc.at[idxr.at[base - 2 + b]],
                                        ossem).wait()

                            @pl.when(r + 1 < _NR)
                            def _(obufs=obufs, ogsem=ogsem):
                                for b in range(2):
                                    pltpu.async_copy(
                                        xh.at[idxc.at[base + 2 + b]],
                                        obufs[b], ogsem)
                    return carry
                lax.fori_loop(0, _NR, _round, 0)
                for b in range(2):
                    pltpu.make_async_copy(
                        quads[(_NR - 1) % 2][0][b],
                        acc.at[idxr.at[2 * (_NR - 1) + b]],
                        quads[(_NR - 1) % 2][2]).wait()

        plsc.subcore_barrier()

        @pl.when(cid == 0)
        def _(olo=olo):
            pltpu.sync_copy(acc.at[rows], olo.at[rows])

        @pl.when(cid == 1)
        def _(ohi=ohi):
            pltpu.sync_copy(acc.at[rows], ohi.at[rows])

        if set_idx == 0:
            pltpu.sync_copy(z64.at[rows], acc.at[rows])
            plsc.subcore_barrier()

    @pl.when(cid == 0)
    def _():
        pltpu.sync_copy(dacc.at[rows], dg0.at[rows])

    @pl.when(cid == 1)
    def _():
        pltpu.sync_copy(dacc.at[rows], dg1.at[rows])


@functools.cache
def _build_spmm_pair():
    return pl.kernel(
        _spmm_body,
        out_type=[jax.ShapeDtypeStruct((_NP, _DH), jnp.float32)] * 4
                 + [jax.ShapeDtypeStruct((_NP, _DEGW), jnp.float32)] * 2,
        mesh=plsc.VectorSubcoreMesh(core_axis_name="c", subcore_axis_name="s",
                                    num_cores=_NC, num_subcores=_NS),
        scratch_types=[
            pltpu.VMEM((_NCHUNK, _CH), jnp.int32),
            pltpu.VMEM((_NCHUNK, _CH), jnp.int32),
            pltpu.VMEM((_NCHUNK, _CH), jnp.int32),
            pltpu.VMEM((_NCHUNK, _CH), jnp.int32),
            pltpu.VMEM((_CH, _DH), jnp.float32),
            pltpu.VMEM((_CH, _DH), jnp.float32),
            pltpu.VMEM((_CH, _DH), jnp.float32),
            pltpu.VMEM((_CH, _DH), jnp.float32),
            pltpu.VMEM((_CH, _DEGW), jnp.float32),
            pltpu.VMEM_SHARED((_NP, _DH), jnp.float32),
            pltpu.VMEM_SHARED((_NP, _DEGW), jnp.float32),
            pltpu.SemaphoreType.DMA,
            pltpu.SemaphoreType.DMA,
            pltpu.SemaphoreType.DMA,
            pltpu.SemaphoreType.DMA,
        ],
        compiler_params=pltpu.CompilerParams(use_tc_tiling_on_sc=False),
    )


def _spmm_pair(*args):
    return _build_spmm_pair()(*args)


_R = 400


def _mlp_block(att_ref, w1lo_ref, w1hi_ref, b1_ref, w2_ref, b2_ref,
               a0lo_ref, a0hi_ref, a1lo_ref, a1hi_ref, d0_ref, d1_ref,
               olo_ref, ohi_ref):
    a = att_ref[...][0]
    e = jnp.exp(a - jnp.max(a))
    w = e / jnp.sum(e)
    deg0 = d0_ref[:, 0:1]
    deg1 = d1_ref[:, 0:1]
    s0 = w[0] / jnp.maximum(deg0, 1.0)
    s1 = w[1] / jnp.maximum(deg1, 1.0)
    glo = a0lo_ref[...] * s0 + a1lo_ref[...] * s1
    ghi = a0hi_ref[...] * s0 + a1hi_ref[...] * s1
    t = jnp.dot(glo, w1lo_ref[...], preferred_element_type=jnp.float32)
    t = t + jnp.dot(ghi, w1hi_ref[...], preferred_element_type=jnp.float32)
    m = w[0] * jnp.minimum(deg0, 1.0) + w[1] * jnp.minimum(deg1, 1.0)
    t = t + m * b1_ref[...]
    h = jnp.where(t >= 0, t, 0.2 * t)
    s2 = jnp.dot(h, w2_ref[...], preferred_element_type=jnp.float32)
    s2 = s2 + b2_ref[...]
    olo_ref[...] = s2[:, :_DH]
    ohi_ref[...] = s2[:, _DH:]


def _norm_block(att_ref, v0lo_ref, v0hi_ref, v1lo_ref, v1hi_ref,
                d0_ref, d1_ref, out_ref):
    a = att_ref[...][1]
    e = jnp.exp(a - jnp.max(a))
    w = e / jnp.sum(e)
    s0 = w[0] / jnp.maximum(d0_ref[:, 0:1], 1.0)
    s1 = w[1] / jnp.maximum(d1_ref[:, 0:1], 1.0)
    olo = v0lo_ref[...] * s0 + v1lo_ref[...] * s1
    ohi = v0hi_ref[...] * s0 + v1hi_ref[...] * s1
    nsq = (jnp.sum(olo * olo, axis=1, keepdims=True)
           + jnp.sum(ohi * ohi, axis=1, keepdims=True))
    inv = 1.0 / jnp.maximum(jnp.sqrt(nsq), 1e-12)
    out_ref[...] = jnp.concatenate([olo * inv, ohi * inv], axis=1)


def _row_spec(width):
    return pl.BlockSpec((_R, width), lambda i: (i, 0))


def _const_spec(shape):
    return pl.BlockSpec(shape, lambda i: tuple(0 for _ in shape))


def kernel(x, edge_a0, edge_a1, edge_r0, edge_r1, W1, b1, W2, b2, att):
    x = x.astype(jnp.float32)
    xlo = x[:, :_DH]
    xhi = x[:, _DH:]

    def _split(e):
        e = e.astype(jnp.int32)
        return (e[0].reshape(_E // _CH, _CH), e[1].reshape(_E // _CH, _CH))

    ra0, ca0 = _split(edge_a0)
    ra1, ca1 = _split(edge_a1)
    rr0, cr0 = _split(edge_r0)
    rr1, cr1 = _split(edge_r1)
    z64 = jnp.zeros((_NP, _DH), jnp.float32)
    zdeg = jnp.zeros((_NP, _DEGW), jnp.float32)

    a0lo, a0hi, a1lo, a1hi, dga0, dga1 = _spmm_pair(
        xlo, xhi, ra0, ca0, ra1, ca1, z64, zdeg)

    w1lo = W1[:_DH, :]
    w1hi = W1[_DH:, :]
    s2lo, s2hi = pl.pallas_call(
        _mlp_block,
        grid=(_N // _R,),
        in_specs=[
            _const_spec((2, 2)),
            _const_spec((_DH, _D_HID)),
            _const_spec((_DH, _D_HID)),
            _const_spec((1, _D_HID)),
            _const_spec((_D_HID, _D_OUT)),
            _const_spec((1, _D_OUT)),
            _row_spec(_DH), _row_spec(_DH), _row_spec(_DH), _row_spec(_DH),
            _row_spec(_DEGW), _row_spec(_DEGW),
        ],
        out_specs=[_row_spec(_DH), _row_spec(_DH)],
        out_shape=[jax.ShapeDtypeStruct((_N, _DH), jnp.float32)] * 2,
    )(att, w1lo, w1hi, b1.reshape(1, _D_HID), W2, b2.reshape(1, _D_OUT),
      a0lo, a0hi, a1lo, a1hi, dga0, dga1)

    v0lo, v0hi, v1lo, v1hi, dgr0, dgr1 = _spmm_pair(
        s2lo, s2hi, rr0, cr0, rr1, cr1, z64, zdeg)

    out = pl.pallas_call(
        _norm_block,
        grid=(_N // _R,),
        in_specs=[
            _const_spec((2, 2)),
            _row_spec(_DH), _row_spec(_DH), _row_spec(_DH), _row_spec(_DH),
            _row_spec(_DEGW), _row_spec(_DEGW),
        ],
        out_specs=_row_spec(_D_OUT),
        out_shape=jax.ShapeDtypeStruct((_N, _D_OUT), jnp.float32),
    )(att, v0lo, v0hi, v1lo, v1hi, dgr0, dgr1)
    return out


# speedup vs baseline: 1.1731x; 1.1731x over previous
import functools

import jax
import jax.numpy as jnp
from jax import lax
from jax.experimental import pallas as pl
from jax.experimental.pallas import tpu as pltpu
from jax.experimental.pallas import tpu_sc as plsc

_N = 10000
_E = 160000
_D_IN = 128
_D_HID = 256
_D_OUT = 128
_DH = 64
_NC, _NS = 2, 16
_NP = 10240
_RPT = _NP // _NS
_EPT = _E // _NS
_CH = 250
_NCHUNK = _EPT // _CH
_DEGW = 16


def _spmm_body(xlo, xhi, r0, c0, r1, c1, z64, zdeg,
               o0lo, o0hi, o1lo, o1hi, dg0, dg1,
               idxr0, idxc0, idxr1, idxc1, gbuf0, gbuf1, ones,
               acc, dacc, sem0, sem1):
    cid = lax.axis_index("c")
    sid = lax.axis_index("s")
    rows = pl.ds(sid * _RPT, _RPT)

    def _ones_row(j, carry):
        ones[j, :] = jnp.ones((16,), jnp.float32)
        return carry
    lax.fori_loop(0, _CH, _ones_row, 0)

    pltpu.sync_copy(z64.at[rows], acc.at[rows])
    pltpu.sync_copy(zdeg.at[rows], dacc.at[rows])

    echunks = pl.ds(sid * _NCHUNK, _NCHUNK)
    pltpu.sync_copy(r0.at[echunks], idxr0)
    pltpu.sync_copy(c0.at[echunks], idxc0)
    pltpu.sync_copy(r1.at[echunks], idxr1)
    pltpu.sync_copy(c1.at[echunks], idxc1)
    plsc.subcore_barrier()

    for set_idx, (idxr, idxc, olo, ohi) in enumerate(
            ((idxr0, idxc0, o0lo, o0hi), (idxr1, idxc1, o1lo, o1hi))):
        for half, xh in ((0, xlo), (1, xhi)):
            @pl.when(cid == half)
            def _(xh=xh, idxr=idxr, idxc=idxc, set_idx=set_idx):
                pltpu.async_copy(xh.at[idxc.at[0]], gbuf0, sem0)
                pltpu.async_copy(xh.at[idxc.at[1]], gbuf1, sem1)

                def _pair(j2, carry):
                    j = 2 * j2
                    for b, gb, sm in ((0, gbuf0, sem0), (1, gbuf1, sem1)):
                        jj = j + b
                        pltpu.make_async_copy(
                            xh.at[idxc.at[jj]], gb, sm).wait()
                        pltpu.sync_copy(gb, acc.at[idxr.at[jj]], add=True)

                        @pl.when(j2 + 1 < _NCHUNK // 2)
                        def _(jj=jj, gb=gb, sm=sm):
                            pltpu.async_copy(xh.at[idxc.at[jj + 2]], gb, sm)

                        @pl.when(cid == set_idx)
                        def _(jj=jj, idxr=idxr):
                            pltpu.sync_copy(ones, dacc.at[idxr.at[jj]],
                                            add=True)
                    return carry
                lax.fori_loop(0, _NCHUNK // 2, _pair, 0)

        plsc.subcore_barrier()

        @pl.when(cid == 0)
        def _(olo=olo):
            pltpu.sync_copy(acc.at[rows], olo.at[rows])

        @pl.when(cid == 1)
        def _(ohi=ohi):
            pltpu.sync_copy(acc.at[rows], ohi.at[rows])

        if set_idx == 0:
            pltpu.sync_copy(z64.at[rows], acc.at[rows])
            plsc.subcore_barrier()

    @pl.when(cid == 0)
    def _():
        pltpu.sync_copy(dacc.at[rows], dg0.at[rows])

    @pl.when(cid == 1)
    def _():
        pltpu.sync_copy(dacc.at[rows], dg1.at[rows])


@functools.cache
def _build_spmm_pair():
    return pl.kernel(
        _spmm_body,
        out_type=[jax.ShapeDtypeStruct((_NP, _DH), jnp.float32)] * 4
                 + [jax.ShapeDtypeStruct((_NP, _DEGW), jnp.float32)] * 2,
        mesh=plsc.VectorSubcoreMesh(core_axis_name="c", subcore_axis_name="s",
                                    num_cores=_NC, num_subcores=_NS),
        scratch_types=[
            pltpu.VMEM((_NCHUNK, _CH), jnp.int32),
            pltpu.VMEM((_NCHUNK, _CH), jnp.int32),
            pltpu.VMEM((_NCHUNK, _CH), jnp.int32),
            pltpu.VMEM((_NCHUNK, _CH), jnp.int32),
            pltpu.VMEM((_CH, _DH), jnp.float32),
            pltpu.VMEM((_CH, _DH), jnp.float32),
            pltpu.VMEM((_CH, _DEGW), jnp.float32),
            pltpu.VMEM_SHARED((_NP, _DH), jnp.float32),
            pltpu.VMEM_SHARED((_NP, _DEGW), jnp.float32),
            pltpu.SemaphoreType.DMA,
            pltpu.SemaphoreType.DMA,
        ],
        compiler_params=pltpu.CompilerParams(use_tc_tiling_on_sc=False),
    )


def _spmm_pair(*args):
    return _build_spmm_pair()(*args)


_R = 400


def _mlp_block(att_ref, w1lo_ref, w1hi_ref, b1_ref, w2_ref, b2_ref,
               a0lo_ref, a0hi_ref, a1lo_ref, a1hi_ref, d0_ref, d1_ref,
               olo_ref, ohi_ref):
    a = att_ref[...][0]
    e = jnp.exp(a - jnp.max(a))
    w = e / jnp.sum(e)
    deg0 = d0_ref[:, 0:1]
    deg1 = d1_ref[:, 0:1]
    s0 = w[0] / jnp.maximum(deg0, 1.0)
    s1 = w[1] / jnp.maximum(deg1, 1.0)
    glo = a0lo_ref[...] * s0 + a1lo_ref[...] * s1
    ghi = a0hi_ref[...] * s0 + a1hi_ref[...] * s1
    t = jnp.dot(glo, w1lo_ref[...], preferred_element_type=jnp.float32)
    t = t + jnp.dot(ghi, w1hi_ref[...], preferred_element_type=jnp.float32)
    m = w[0] * jnp.minimum(deg0, 1.0) + w[1] * jnp.minimum(deg1, 1.0)
    t = t + m * b1_ref[...]
    h = jnp.where(t >= 0, t, 0.2 * t)
    s2 = jnp.dot(h, w2_ref[...], preferred_element_type=jnp.float32)
    s2 = s2 + b2_ref[...]
    olo_ref[...] = s2[:, :_DH]
    ohi_ref[...] = s2[:, _DH:]


def _norm_block(att_ref, v0lo_ref, v0hi_ref, v1lo_ref, v1hi_ref,
                d0_ref, d1_ref, out_ref):
    a = att_ref[...][1]
    e = jnp.exp(a - jnp.max(a))
    w = e / jnp.sum(e)
    s0 = w[0] / jnp.maximum(d0_ref[:, 0:1], 1.0)
    s1 = w[1] / jnp.maximum(d1_ref[:, 0:1], 1.0)
    olo = v0lo_ref[...] * s0 + v1lo_ref[...] * s1
    ohi = v0hi_ref[...] * s0 + v1hi_ref[...] * s1
    nsq = (jnp.sum(olo * olo, axis=1, keepdims=True)
           + jnp.sum(ohi * ohi, axis=1, keepdims=True))
    inv = 1.0 / jnp.maximum(jnp.sqrt(nsq), 1e-12)
    out_ref[...] = jnp.concatenate([olo * inv, ohi * inv], axis=1)


def _row_spec(width):
    return pl.BlockSpec((_R, width), lambda i: (i, 0))


def _const_spec(shape):
    return pl.BlockSpec(shape, lambda i: tuple(0 for _ in shape))


def kernel(x, edge_a0, edge_a1, edge_r0, edge_r1, W1, b1, W2, b2, att):
    x = x.astype(jnp.float32)
    xlo = x[:, :_DH]
    xhi = x[:, _DH:]

    def _split(e):
        e = e.astype(jnp.int32)
        return (e[0].reshape(_E // _CH, _CH), e[1].reshape(_E // _CH, _CH))

    ra0, ca0 = _split(edge_a0)
    ra1, ca1 = _split(edge_a1)
    rr0, cr0 = _split(edge_r0)
    rr1, cr1 = _split(edge_r1)
    z64 = jnp.zeros((_NP, _DH), jnp.float32)
    zdeg = jnp.zeros((_NP, _DEGW), jnp.float32)

    a0lo, a0hi, a1lo, a1hi, dga0, dga1 = _spmm_pair(
        xlo, xhi, ra0, ca0, ra1, ca1, z64, zdeg)

    w1lo = W1[:_DH, :]
    w1hi = W1[_DH:, :]
    s2lo, s2hi = pl.pallas_call(
        _mlp_block,
        grid=(_N // _R,),
        in_specs=[
            _const_spec((2, 2)),
            _const_spec((_DH, _D_HID)),
            _const_spec((_DH, _D_HID)),
            _const_spec((1, _D_HID)),
            _const_spec((_D_HID, _D_OUT)),
            _const_spec((1, _D_OUT)),
            _row_spec(_DH), _row_spec(_DH), _row_spec(_DH), _row_spec(_DH),
            _row_spec(_DEGW), _row_spec(_DEGW),
        ],
        out_specs=[_row_spec(_DH), _row_spec(_DH)],
        out_shape=[jax.ShapeDtypeStruct((_N, _DH), jnp.float32)] * 2,
    )(att, w1lo, w1hi, b1.reshape(1, _D_HID), W2, b2.reshape(1, _D_OUT),
      a0lo, a0hi, a1lo, a1hi, dga0, dga1)

    v0lo, v0hi, v1lo, v1hi, dgr0, dgr1 = _spmm_pair(
        s2lo, s2hi, rr0, cr0, rr1, cr1, z64, zdeg)

    out = pl.pallas_call(
        _norm_block,
        grid=(_N // _R,),
        in_specs=[
            _const_spec((2, 2)),
            _row_spec(_DH), _row_spec(_DH), _row_spec(_DH), _row_spec(_DH),
            _row_spec(_DEGW), _row_spec(_DEGW),
        ],
        out_specs=_row_spec(_D_OUT),
        out_shape=jax.ShapeDtypeStruct((_N, _D_OUT), jnp.float32),
    )(att, v0lo, v0hi, v1lo, v1hi, dgr0, dgr1)
    return out


# speedup vs baseline: 1.2456x; 1.0618x over previous
import functools

import jax
import jax.numpy as jnp
from jax import lax
from jax.experimental import pallas as pl
from jax.experimental.pallas import tpu as pltpu
from jax.experimental.pallas import tpu_sc as plsc

_N = 10000
_E = 160000
_D_IN = 128
_D_HID = 256
_D_OUT = 128
_DH = 64
_NC, _NS = 2, 16
_NP = 10240
_RPT = _NP // _NS
_EPT = _E // _NS
_CH = 250
_NCHUNK = _EPT // _CH
_DEGW = 16


def _spmm_body(xlo, xhi, r0, c0, r1, c1, z64, zdeg,
               o0lo, o0hi, o1lo, o1hi, dg0, dg1,
               idxr0, idxc0, idxr1, idxc1, gbuf0, gbuf1, ones,
               acc, dacc, sem0, sem1):
    cid = lax.axis_index("c")
    sid = lax.axis_index("s")
    rows = pl.ds(sid * _RPT, _RPT)

    def _ones_row(j, carry):
        ones[j, :] = jnp.ones((16,), jnp.float32)
        return carry
    lax.fori_loop(0, _CH, _ones_row, 0)

    pltpu.sync_copy(z64.at[rows], acc.at[rows])
    pltpu.sync_copy(zdeg.at[rows], dacc.at[rows])

    echunks = pl.ds(sid * _NCHUNK, _NCHUNK)
    pltpu.sync_copy(r0.at[echunks], idxr0)
    pltpu.sync_copy(c0.at[echunks], idxc0)
    pltpu.sync_copy(r1.at[echunks], idxr1)
    pltpu.sync_copy(c1.at[echunks], idxc1)
    plsc.subcore_barrier()

    for set_idx, (idxr, idxc, olo, ohi) in enumerate(
            ((idxr0, idxc0, o0lo, o0hi), (idxr1, idxc1, o1lo, o1hi))):
        for half, xh in ((0, xlo), (1, xhi)):
            @pl.when(cid == half)
            def _(xh=xh, idxr=idxr, idxc=idxc, set_idx=set_idx):
                pltpu.async_copy(xh.at[idxc.at[0]], gbuf0, sem0)
                pltpu.async_copy(xh.at[idxc.at[1]], gbuf1, sem1)

                def _pair(j2, carry):
                    j = 2 * j2
                    for b, gb, sm in ((0, gbuf0, sem0), (1, gbuf1, sem1)):
                        jj = j + b
                        pltpu.make_async_copy(
                            xh.at[idxc.at[jj]], gb, sm).wait()
                        pltpu.sync_copy(gb, acc.at[idxr.at[jj]], add=True)

                        @pl.when(j2 + 1 < _NCHUNK // 2)
                        def _(jj=jj, gb=gb, sm=sm):
                            pltpu.async_copy(xh.at[idxc.at[jj + 2]], gb, sm)

                        @pl.when(cid == set_idx)
                        def _(jj=jj, idxr=idxr):
                            pltpu.sync_copy(ones, dacc.at[idxr.at[jj]],
                                            add=True)
                    return carry
                lax.fori_loop(0, _NCHUNK // 2, _pair, 0)

        plsc.subcore_barrier()

        @pl.when(cid == 0)
        def _(olo=olo):
            pltpu.sync_copy(acc.at[rows], olo.at[rows])

        @pl.when(cid == 1)
        def _(ohi=ohi):
            pltpu.sync_copy(acc.at[rows], ohi.at[rows])

        if set_idx == 0:
            pltpu.sync_copy(z64.at[rows], acc.at[rows])
            plsc.subcore_barrier()

    @pl.when(cid == 0)
    def _():
        pltpu.sync_copy(dacc.at[rows], dg0.at[rows])

    @pl.when(cid == 1)
    def _():
        pltpu.sync_copy(dacc.at[rows], dg1.at[rows])


@functools.cache
def _build_spmm_pair():
    return pl.kernel(
        _spmm_body,
        out_type=[jax.ShapeDtypeStruct((_NP, _DH), jnp.float32)] * 4
                 + [jax.ShapeDtypeStruct((_NP, _DEGW), jnp.float32)] * 2,
        mesh=plsc.VectorSubcoreMesh(core_axis_name="c", subcore_axis_name="s",
                                    num_cores=_NC, num_subcores=_NS),
        scratch_types=[
            pltpu.VMEM((_NCHUNK, _CH), jnp.int32),
            pltpu.VMEM((_NCHUNK, _CH), jnp.int32),
            pltpu.VMEM((_NCHUNK, _CH), jnp.int32),
            pltpu.VMEM((_NCHUNK, _CH), jnp.int32),
            pltpu.VMEM((_CH, _DH), jnp.float32),
            pltpu.VMEM((_CH, _DH), jnp.float32),
            pltpu.VMEM((_CH, _DEGW), jnp.float32),
            pltpu.VMEM_SHARED((_NP, _DH), jnp.float32),
            pltpu.VMEM_SHARED((_NP, _DEGW), jnp.float32),
            pltpu.SemaphoreType.DMA,
            pltpu.SemaphoreType.DMA,
        ],
        compiler_params=pltpu.CompilerParams(use_tc_tiling_on_sc=False),
    )


def _spmm_pair(*args):
    return _build_spmm_pair()(*args)


_R = 1000


def _mlp_block(att_ref, w1lo_ref, w1hi_ref, b1_ref, w2_ref, b2_ref,
               a0lo_ref, a0hi_ref, a1lo_ref, a1hi_ref, d0_ref, d1_ref,
               olo_ref, ohi_ref):
    a = att_ref[...][0]
    e = jnp.exp(a - jnp.max(a))
    w = e / jnp.sum(e)
    deg0 = d0_ref[:, 0:1]
    deg1 = d1_ref[:, 0:1]
    s0 = w[0] / jnp.maximum(deg0, 1.0)
    s1 = w[1] / jnp.maximum(deg1, 1.0)
    glo = a0lo_ref[...] * s0 + a1lo_ref[...] * s1
    ghi = a0hi_ref[...] * s0 + a1hi_ref[...] * s1
    t = jnp.dot(glo, w1lo_ref[...], preferred_element_type=jnp.float32)
    t = t + jnp.dot(ghi, w1hi_ref[...], preferred_element_type=jnp.float32)
    m = w[0] * jnp.minimum(deg0, 1.0) + w[1] * jnp.minimum(deg1, 1.0)
    t = t + m * b1_ref[...]
    h = jnp.where(t >= 0, t, 0.2 * t)
    s2 = jnp.dot(h, w2_ref[...], preferred_element_type=jnp.float32)
    s2 = s2 + b2_ref[...]
    olo_ref[...] = s2[:, :_DH]
    ohi_ref[...] = s2[:, _DH:]


def _norm_block(att_ref, v0lo_ref, v0hi_ref, v1lo_ref, v1hi_ref,
                d0_ref, d1_ref, out_ref):
    a = att_ref[...][1]
    e = jnp.exp(a - jnp.max(a))
    w = e / jnp.sum(e)
    s0 = w[0] / jnp.maximum(d0_ref[:, 0:1], 1.0)
    s1 = w[1] / jnp.maximum(d1_ref[:, 0:1], 1.0)
    olo = v0lo_ref[...] * s0 + v1lo_ref[...] * s1
    ohi = v0hi_ref[...] * s0 + v1hi_ref[...] * s1
    nsq = (jnp.sum(olo * olo, axis=1, keepdims=True)
           + jnp.sum(ohi * ohi, axis=1, keepdims=True))
    inv = 1.0 / jnp.maximum(jnp.sqrt(nsq), 1e-12)
    out_ref[...] = jnp.concatenate([olo * inv, ohi * inv], axis=1)


def _row_spec(width):
    return pl.BlockSpec((_R, width), lambda i: (i, 0))


def _const_spec(shape):
    return pl.BlockSpec(shape, lambda i: tuple(0 for _ in shape))


def kernel(x, edge_a0, edge_a1, edge_r0, edge_r1, W1, b1, W2, b2, att):
    x = x.astype(jnp.float32)
    xlo = x[:, :_DH]
    xhi = x[:, _DH:]

    def _split(e):
        e = e.astype(jnp.int32)
        return (e[0].reshape(_E // _CH, _CH), e[1].reshape(_E // _CH, _CH))

    ra0, ca0 = _split(edge_a0)
    ra1, ca1 = _split(edge_a1)
    rr0, cr0 = _split(edge_r0)
    rr1, cr1 = _split(edge_r1)
    z64 = jnp.zeros((_NP, _DH), jnp.float32)
    zdeg = jnp.zeros((_NP, _DEGW), jnp.float32)

    a0lo, a0hi, a1lo, a1hi, dga0, dga1 = _spmm_pair(
        xlo, xhi, ra0, ca0, ra1, ca1, z64, zdeg)

    w1lo = W1[:_DH, :]
    w1hi = W1[_DH:, :]
    s2lo, s2hi = pl.pallas_call(
        _mlp_block,
        grid=(_N // _R,),
        in_specs=[
            _const_spec((2, 2)),
            _const_spec((_DH, _D_HID)),
            _const_spec((_DH, _D_HID)),
            _const_spec((1, _D_HID)),
            _const_spec((_D_HID, _D_OUT)),
            _const_spec((1, _D_OUT)),
            _row_spec(_DH), _row_spec(_DH), _row_spec(_DH), _row_spec(_DH),
            _row_spec(_DEGW), _row_spec(_DEGW),
        ],
        out_specs=[_row_spec(_DH), _row_spec(_DH)],
        out_shape=[jax.ShapeDtypeStruct((_N, _DH), jnp.float32)] * 2,
    )(att, w1lo, w1hi, b1.reshape(1, _D_HID), W2, b2.reshape(1, _D_OUT),
      a0lo, a0hi, a1lo, a1hi, dga0, dga1)

    v0lo, v0hi, v1lo, v1hi, dgr0, dgr1 = _spmm_pair(
        s2lo, s2hi, rr0, cr0, rr1, cr1, z64, zdeg)

    out = pl.pallas_call(
        _norm_block,
        grid=(_N // _R,),
        in_specs=[
            _const_spec((2, 2)),
            _row_spec(_DH), _row_spec(_DH), _row_spec(_DH), _row_spec(_DH),
            _row_spec(_DEGW), _row_spec(_DEGW),
        ],
        out_specs=_row_spec(_D_OUT),
        out_shape=jax.ShapeDtypeStruct((_N, _D_OUT), jnp.float32),
    )(att, v0lo, v0hi, v1lo, v1hi, dgr0, dgr1)
    return out


# speedup vs baseline: 1.2588x; 1.0105x over previous
import functools

import jax
import jax.numpy as jnp
from jax import lax
from jax.experimental import pallas as pl
from jax.experimental.pallas import tpu as pltpu
from jax.experimental.pallas import tpu_sc as plsc

_N = 10000
_E = 160000
_D_IN = 128
_D_HID = 256
_D_OUT = 128
_DH = 64
_NC, _NS = 2, 16
_NP = 10240
_RPT = _NP // _NS
_EPT = _E // _NS
_CH = 250
_NCHUNK = _EPT // _CH
_DEGW = 16


def _spmm_body(xlo, xhi, r0, c0, r1, c1, z64, zdeg,
               o0lo, o0hi, o1lo, o1hi, dg0, dg1,
               idxr0, idxc0, idxr1, idxc1, gbuf0, gbuf1, ones,
               acc, dacc, sem0, sem1):
    cid = lax.axis_index("c")
    sid = lax.axis_index("s")
    rows = pl.ds(sid * _RPT, _RPT)

    def _ones_row(j, carry):
        ones[j, :] = jnp.ones((16,), jnp.float32)
        return carry
    lax.fori_loop(0, _CH, _ones_row, 0)

    pltpu.sync_copy(z64.at[rows], acc.at[rows])
    pltpu.sync_copy(zdeg.at[rows], dacc.at[rows])

    echunks = pl.ds(sid * _NCHUNK, _NCHUNK)
    pltpu.sync_copy(r0.at[echunks], idxr0)
    pltpu.sync_copy(c0.at[echunks], idxc0)
    pltpu.sync_copy(r1.at[echunks], idxr1)
    pltpu.sync_copy(c1.at[echunks], idxc1)
    plsc.subcore_barrier()

    for set_idx, (idxr, idxc, olo, ohi) in enumerate(
            ((idxr0, idxc0, o0lo, o0hi), (idxr1, idxc1, o1lo, o1hi))):
        for half, xh in ((0, xlo), (1, xhi)):
            @pl.when(cid == half)
            def _(xh=xh, idxr=idxr, idxc=idxc, set_idx=set_idx):
                pltpu.async_copy(xh.at[idxc.at[0]], gbuf0, sem0)
                pltpu.async_copy(xh.at[idxc.at[1]], gbuf1, sem1)

                def _pair(j2, carry):
                    j = 2 * j2
                    for b, gb, sm in ((0, gbuf0, sem0), (1, gbuf1, sem1)):
                        jj = j + b
                        pltpu.make_async_copy(
                            xh.at[idxc.at[jj]], gb, sm).wait()
                        pltpu.sync_copy(gb, acc.at[idxr.at[jj]], add=True)

                        @pl.when(j2 + 1 < _NCHUNK // 2)
                        def _(jj=jj, gb=gb, sm=sm):
                            pltpu.async_copy(xh.at[idxc.at[jj + 2]], gb, sm)

                        @pl.when(cid == set_idx)
                        def _(jj=jj, idxr=idxr):
                            pltpu.sync_copy(ones, dacc.at[idxr.at[jj]],
                                            add=True)
                    return carry
                lax.fori_loop(0, _NCHUNK // 2, _pair, 0)

        plsc.subcore_barrier()

        @pl.when(cid == 0)
        def _(olo=olo):
            pltpu.sync_copy(acc.at[rows], olo.at[rows])

        @pl.when(cid == 1)
        def _(ohi=ohi):
            pltpu.sync_copy(acc.at[rows], ohi.at[rows])

        if set_idx == 0:
            pltpu.sync_copy(z64.at[rows], acc.at[rows])
            plsc.subcore_barrier()

    @pl.when(cid == 0)
    def _():
        pltpu.sync_copy(dacc.at[rows], dg0.at[rows])

    @pl.when(cid == 1)
    def _():
        pltpu.sync_copy(dacc.at[rows], dg1.at[rows])


@functools.cache
def _build_spmm_pair():
    return pl.kernel(
        _spmm_body,
        out_type=[jax.ShapeDtypeStruct((_NP, _DH), jnp.float32)] * 4
                 + [jax.ShapeDtypeStruct((_NP, _DEGW), jnp.float32)] * 2,
        mesh=plsc.VectorSubcoreMesh(core_axis_name="c", subcore_axis_name="s",
                                    num_cores=_NC, num_subcores=_NS),
        scratch_types=[
            pltpu.VMEM((_NCHUNK, _CH), jnp.int32),
            pltpu.VMEM((_NCHUNK, _CH), jnp.int32),
            pltpu.VMEM((_NCHUNK, _CH), jnp.int32),
            pltpu.VMEM((_NCHUNK, _CH), jnp.int32),
            pltpu.VMEM((_CH, _DH), jnp.float32),
            pltpu.VMEM((_CH, _DH), jnp.float32),
            pltpu.VMEM((_CH, _DEGW), jnp.float32),
            pltpu.VMEM_SHARED((_NP, _DH), jnp.float32),
            pltpu.VMEM_SHARED((_NP, _DEGW), jnp.float32),
            pltpu.SemaphoreType.DMA,
            pltpu.SemaphoreType.DMA,
        ],
        compiler_params=pltpu.CompilerParams(use_tc_tiling_on_sc=False),
    )


def _spmm_pair(*args):
    return _build_spmm_pair()(*args)


_R = 2000


def _mlp_block(att_ref, w1lo_ref, w1hi_ref, b1_ref, w2_ref, b2_ref,
               a0lo_ref, a0hi_ref, a1lo_ref, a1hi_ref, d0_ref, d1_ref,
               olo_ref, ohi_ref):
    a = att_ref[...][0]
    e = jnp.exp(a - jnp.max(a))
    w = e / jnp.sum(e)
    deg0 = d0_ref[:, 0:1]
    deg1 = d1_ref[:, 0:1]
    s0 = w[0] / jnp.maximum(deg0, 1.0)
    s1 = w[1] / jnp.maximum(deg1, 1.0)
    glo = a0lo_ref[...] * s0 + a1lo_ref[...] * s1
    ghi = a0hi_ref[...] * s0 + a1hi_ref[...] * s1
    t = jnp.dot(glo, w1lo_ref[...], preferred_element_type=jnp.float32)
    t = t + jnp.dot(ghi, w1hi_ref[...], preferred_element_type=jnp.float32)
    m = w[0] * jnp.minimum(deg0, 1.0) + w[1] * jnp.minimum(deg1, 1.0)
    t = t + m * b1_ref[...]
    h = jnp.where(t >= 0, t, 0.2 * t)
    s2 = jnp.dot(h, w2_ref[...], preferred_element_type=jnp.float32)
    s2 = s2 + b2_ref[...]
    olo_ref[...] = s2[:, :_DH]
    ohi_ref[...] = s2[:, _DH:]


def _norm_block(att_ref, v0lo_ref, v0hi_ref, v1lo_ref, v1hi_ref,
                d0_ref, d1_ref, out_ref):
    a = att_ref[...][1]
    e = jnp.exp(a - jnp.max(a))
    w = e / jnp.sum(e)
    s0 = w[0] / jnp.maximum(d0_ref[:, 0:1], 1.0)
    s1 = w[1] / jnp.maximum(d1_ref[:, 0:1], 1.0)
    olo = v0lo_ref[...] * s0 + v1lo_ref[...] * s1
    ohi = v0hi_ref[...] * s0 + v1hi_ref[...] * s1
    nsq = (jnp.sum(olo * olo, axis=1, keepdims=True)
           + jnp.sum(ohi * ohi, axis=1, keepdims=True))
    inv = 1.0 / jnp.maximum(jnp.sqrt(nsq), 1e-12)
    out_ref[...] = jnp.concatenate([olo * inv, ohi * inv], axis=1)


def _row_spec(width):
    return pl.BlockSpec((_R, width), lambda i: (i, 0))


def _const_spec(shape):
    return pl.BlockSpec(shape, lambda i: tuple(0 for _ in shape))


def kernel(x, edge_a0, edge_a1, edge_r0, edge_r1, W1, b1, W2, b2, att):
    x = x.astype(jnp.float32)
    xlo = x[:, :_DH]
    xhi = x[:, _DH:]

    def _split(e):
        e = e.astype(jnp.int32)
        return (e[0].reshape(_E // _CH, _CH), e[1].reshape(_E // _CH, _CH))

    ra0, ca0 = _split(edge_a0)
    ra1, ca1 = _split(edge_a1)
    rr0, cr0 = _split(edge_r0)
    rr1, cr1 = _split(edge_r1)
    z64 = jnp.zeros((_NP, _DH), jnp.float32)
    zdeg = jnp.zeros((_NP, _DEGW), jnp.float32)

    a0lo, a0hi, a1lo, a1hi, dga0, dga1 = _spmm_pair(
        xlo, xhi, ra0, ca0, ra1, ca1, z64, zdeg)

    w1lo = W1[:_DH, :]
    w1hi = W1[_DH:, :]
    s2lo, s2hi = pl.pallas_call(
        _mlp_block,
        grid=(_N // _R,),
        in_specs=[
            _const_spec((2, 2)),
            _const_spec((_DH, _D_HID)),
            _const_spec((_DH, _D_HID)),
            _const_spec((1, _D_HID)),
            _const_spec((_D_HID, _D_OUT)),
            _const_spec((1, _D_OUT)),
            _row_spec(_DH), _row_spec(_DH), _row_spec(_DH), _row_spec(_DH),
            _row_spec(_DEGW), _row_spec(_DEGW),
        ],
        out_specs=[_row_spec(_DH), _row_spec(_DH)],
        out_shape=[jax.ShapeDtypeStruct((_N, _DH), jnp.float32)] * 2,
    )(att, w1lo, w1hi, b1.reshape(1, _D_HID), W2, b2.reshape(1, _D_OUT),
      a0lo, a0hi, a1lo, a1hi, dga0, dga1)

    v0lo, v0hi, v1lo, v1hi, dgr0, dgr1 = _spmm_pair(
        s2lo, s2hi, rr0, cr0, rr1, cr1, z64, zdeg)

    out = pl.pallas_call(
        _norm_block,
        grid=(_N // _R,),
        in_specs=[
            _const_spec((2, 2)),
            _row_spec(_DH), _row_spec(_DH), _row_spec(_DH), _row_spec(_DH),
            _row_spec(_DEGW), _row_spec(_DEGW),
        ],
        out_specs=_row_spec(_D_OUT),
        out_shape=jax.ShapeDtypeStruct((_N, _D_OUT), jnp.float32),
    )(att, v0lo, v0hi, v1lo, v1hi, dgr0, dgr1)
    return out
